# Initial kernel scaffold; baseline (speedup 1.0000x reference)
import jax
import jax.numpy as jnp
from jax.experimental import pallas as pl


def kernel(inp, edge_index, W1, a_src1, a_dst1, b1, W2, a_src2, a_dst2, b2, W3, a_src3, a_dst3, b3):
    raise NotImplementedError("write your pallas kernel here")


# speedup vs baseline: 36.0748x; 36.0748x over previous
import functools

import jax
import jax.numpy as jnp
from jax import lax
from jax.experimental import pallas as pl
from jax.experimental.pallas import tpu as pltpu
from jax.experimental.pallas import tpu_sc as plsc

N = 10000
E = 320000
IN_C = 128
HID = 64
NEG = 0.2

NC = 2
NS = 16
NW = NC * NS
ET = E // NW
CH = 80
NCHUNK = ET // CH
ROWS = N // NS
RSTG = 125
DEN_R = N // 16

RB = 1000


def _elu(x):
    return jnp.where(x > 0, x, jnp.exp(jnp.minimum(x, 0.0)) - 1.0)


def _project_body(x_ref, w_ref, asv_ref, adv_ref, h_ref, s_ref, d_ref):
    h = jnp.dot(x_ref[...], w_ref[...], preferred_element_type=jnp.float32)
    h_ref[...] = h
    s_ref[...] = jnp.sum(h * asv_ref[...], axis=1, keepdims=True)
    d_ref[...] = jnp.sum(h * adv_ref[...], axis=1, keepdims=True)


def _tc_project(x, W, asv, adv):
    cin = x.shape[1]
    return pl.pallas_call(
        _project_body,
        grid=(N // RB,),
        in_specs=[
            pl.BlockSpec((RB, cin), lambda i: (i, 0)),
            pl.BlockSpec((cin, HID), lambda i: (0, 0)),
            pl.BlockSpec((1, HID), lambda i: (0, 0)),
            pl.BlockSpec((1, HID), lambda i: (0, 0)),
        ],
        out_specs=[
            pl.BlockSpec((RB, HID), lambda i: (i, 0)),
            pl.BlockSpec((RB, 1), lambda i: (i, 0)),
            pl.BlockSpec((RB, 1), lambda i: (i, 0)),
        ],
        out_shape=[
            jax.ShapeDtypeStruct((N, HID), jnp.float32),
            jax.ShapeDtypeStruct((N, 1), jnp.float32),
            jax.ShapeDtypeStruct((N, 1), jnp.float32),
        ],
    )(x, W, asv, adv)


def _finalize_block(a0_ref, a1_ref, den_ref, b_ref):
    den = jnp.sum(den_ref[...], axis=1, keepdims=True)
    x = (a0_ref[...] + a1_ref[...]) / (den + 1e-16) + b_ref[...]
    return _elu(x)


def _finproj_body(a0_ref, a1_ref, den_ref, b_ref, w_ref, asv_ref, adv_ref,
                  h_ref, s_ref, d_ref):
    x = _finalize_block(a0_ref, a1_ref, den_ref, b_ref)
    h = jnp.dot(x, w_ref[...], preferred_element_type=jnp.float32)
    h_ref[...] = h
    s_ref[...] = jnp.sum(h * asv_ref[...], axis=1, keepdims=True)
    d_ref[...] = jnp.sum(h * adv_ref[...], axis=1, keepdims=True)


def _tc_finproj(a0, a1, denT, b, W, asv, adv):
    return pl.pallas_call(
        _finproj_body,
        grid=(N // RB,),
        in_specs=[
            pl.BlockSpec((RB, HID), lambda i: (i, 0)),
            pl.BlockSpec((RB, HID), lambda i: (i, 0)),
            pl.BlockSpec((RB, NW), lambda i: (i, 0)),
            pl.BlockSpec((1, HID), lambda i: (0, 0)),
            pl.BlockSpec((HID, HID), lambda i: (0, 0)),
            pl.BlockSpec((1, HID), lambda i: (0, 0)),
            pl.BlockSpec((1, HID), lambda i: (0, 0)),
        ],
        out_specs=[
            pl.BlockSpec((RB, HID), lambda i: (i, 0)),
            pl.BlockSpec((RB, 1), lambda i: (i, 0)),
            pl.BlockSpec((RB, 1), lambda i: (i, 0)),
        ],
        out_shape=[
            jax.ShapeDtypeStruct((N, HID), jnp.float32),
            jax.ShapeDtypeStruct((N, 1), jnp.float32),
            jax.ShapeDtypeStruct((N, 1), jnp.float32),
        ],
    )(a0, a1, denT, b, W, asv, adv)


def _final_body(a0_ref, a1_ref, den_ref, b_ref, o_ref):
    o_ref[...] = _finalize_block(a0_ref, a1_ref, den_ref, b_ref)


def _tc_final(a0, a1, denT, b):
    return pl.pallas_call(
        _final_body,
        grid=(N // RB,),
        in_specs=[
            pl.BlockSpec((RB, HID), lambda i: (i, 0)),
            pl.BlockSpec((RB, HID), lambda i: (i, 0)),
            pl.BlockSpec((RB, NW), lambda i: (i, 0)),
            pl.BlockSpec((1, HID), lambda i: (0, 0)),
        ],
        out_specs=pl.BlockSpec((RB, HID), lambda i: (i, 0)),
        out_shape=jax.ShapeDtypeStruct((N, HID), jnp.float32),
    )(a0, a1, denT, b)


def _sc_edge_body(h_hbm, asrc_hbm, adst_hbm, src_hbm, dst_hbm,
                  acc_hbm, den_hbm,
                  asrc_v, adst_v, src_v, dst_v, w_v, den_v, rows_v, out_v,
                  out_sh, sem):
    c = lax.axis_index("c")
    s = lax.axis_index("s")
    wid = c * NS + s

    pltpu.sync_copy(asrc_hbm, asrc_v)
    pltpu.sync_copy(adst_hbm, adst_v)
    pltpu.sync_copy(src_hbm.at[wid], src_v)
    pltpu.sync_copy(dst_hbm.at[wid], dst_v)

    zeros16 = jnp.zeros((16,), jnp.float32)

    def _zero_den(i, carry):
        den_v[i, :] = zeros16
        return carry

    lax.fori_loop(0, DEN_R, _zero_den, 0)

    def _zero_out(i, carry):
        for q in range(HID // 16):
            out_v[i, pl.ds(q * 16, 16)] = zeros16
        return carry

    lax.fori_loop(0, RSTG, _zero_out, 0)
    for r in range(ROWS // RSTG):
        pltpu.sync_copy(out_v, out_sh.at[pl.ds(s * ROWS + r * RSTG, RSTG)])
    plsc.subcore_barrier()

    def _wchunk(j, carry):
        for k in range(CH // 16):
            sidx = src_v[j, pl.ds(k * 16, 16)]
            didx = dst_v[j, pl.ds(k * 16, 16)]
            a1 = plsc.load_gather(asrc_v, [sidx])
            a2 = plsc.load_gather(adst_v, [didx])
            e = a1 + a2
            e = jnp.where(e >= 0, e, e * NEG)
            w16 = jnp.exp(e)
            w_v[j, pl.ds(k * 16, 16)] = w16
            plsc.addupdate_scatter(
                den_v, [lax.shift_right_logical(didx, 4), didx & 15], w16)
        return carry

    lax.fori_loop(0, NCHUNK, _wchunk, 0)

    def _vchunk(j, carry):
        pltpu.async_copy(h_hbm.at[src_v.at[j]], rows_v, sem).wait()
        for k in range(CH // 16):
            w16 = w_v[j, pl.ds(k * 16, 16)]
            for l in range(16):
                e = k * 16 + l
                wl = w16[l]
                for q in range(HID // 16):
                    rows_v[e, pl.ds(q * 16, 16)] = (
                        rows_v[e, pl.ds(q * 16, 16)] * wl)
        pltpu.async_copy(rows_v, out_sh.at[dst_v.at[j]], sem, add=True).wait()
        return carry

    lax.fori_loop(0, NCHUNK, _vchunk, 0)
    plsc.subcore_barrier()

    for r in range(ROWS // RSTG):
        pltpu.sync_copy(out_sh.at[pl.ds(s * ROWS + r * RSTG, RSTG)], out_v)
        pltpu.sync_copy(out_v, acc_hbm.at[c, pl.ds(s * ROWS + r * RSTG, RSTG)])
    pltpu.sync_copy(den_v, den_hbm.at[wid])


@functools.partial(
    pl.kernel,
    out_type=(
        pltpu.HBM((NC, N, HID), jnp.float32),
        pltpu.HBM((NW, DEN_R, 16), jnp.float32),
    ),
    mesh=plsc.VectorSubcoreMesh(core_axis_name="c", subcore_axis_name="s"),
    compiler_params=pltpu.CompilerParams(use_tc_tiling_on_sc=False,
                                         needs_layout_passes=False),
    scratch_types=[
        pltpu.VMEM((N,), jnp.float32),
        pltpu.VMEM((N,), jnp.float32),
        pltpu.VMEM((NCHUNK, CH), jnp.int32),
        pltpu.VMEM((NCHUNK, CH), jnp.int32),
        pltpu.VMEM((NCHUNK, CH), jnp.float32),
        pltpu.VMEM((DEN_R, 16), jnp.float32),
        pltpu.VMEM((CH, HID), jnp.float32),
        pltpu.VMEM((RSTG, HID), jnp.float32),
        pltpu.VMEM_SHARED((N, HID), jnp.float32),
        pltpu.SemaphoreType.DMA,
    ],
)
def _sc_edge(h_hbm, asrc_hbm, adst_hbm, src_hbm, dst_hbm, acc_hbm, den_hbm,
             *rest):
    _sc_edge_body(h_hbm, asrc_hbm, adst_hbm, src_hbm, dst_hbm,
                  acc_hbm, den_hbm, *rest)


def kernel(inp, edge_index, W1, a_src1, a_dst1, b1, W2, a_src2, a_dst2, b2,
           W3, a_src3, a_dst3, b3):
    src3 = edge_index[0].reshape(NW, NCHUNK, CH)
    dst3 = edge_index[1].reshape(NW, NCHUNK, CH)

    h, s, d = _tc_project(inp, W1, a_src1.reshape(1, HID),
                          a_dst1.reshape(1, HID))
    acc, den = _sc_edge(h, s.reshape(N), d.reshape(N), src3, dst3)
    denT = den.reshape(NW, N).T

    h, s, d = _tc_finproj(acc[0], acc[1], denT, b1.reshape(1, HID), W2,
                          a_src2.reshape(1, HID), a_dst2.reshape(1, HID))
    acc, den = _sc_edge(h, s.reshape(N), d.reshape(N), src3, dst3)
    denT = den.reshape(NW, N).T

    h, s, d = _tc_finproj(acc[0], acc[1], denT, b2.reshape(1, HID), W3,
                          a_src3.reshape(1, HID), a_dst3.reshape(1, HID))
    acc, den = _sc_edge(h, s.reshape(N), d.reshape(N), src3, dst3)
    denT = den.reshape(NW, N).T

    return _tc_final(acc[0], acc[1], denT, b3.reshape(1, HID))


# speedup vs baseline: 57.5998x; 1.5967x over previous
import functools

import jax
import jax.numpy as jnp
from jax import lax
from jax.experimental import pallas as pl
from jax.experimental.pallas import tpu as pltpu
from jax.experimental.pallas import tpu_sc as plsc

N = 10000
E = 320000
IN_C = 128
HID = 64
NEG = 0.2

NC = 2
NS = 16
NW = NC * NS
ET = E // NW
CH = 80
NCHUNK = ET // CH
ROWS = N // NS
RSTG = 25
CBYTES = CH * HID * 4
DEN_R = N // 16

RB = 1000


def _elu(x):
    return jnp.where(x > 0, x, jnp.exp(jnp.minimum(x, 0.0)) - 1.0)


def _project_body(x_ref, w_ref, asv_ref, adv_ref, h_ref, s_ref, d_ref):
    h = jnp.dot(x_ref[...], w_ref[...], preferred_element_type=jnp.float32)
    h_ref[...] = h
    s_ref[...] = jnp.sum(h * asv_ref[...], axis=1, keepdims=True)
    d_ref[...] = jnp.sum(h * adv_ref[...], axis=1, keepdims=True)


def _tc_project(x, W, asv, adv):
    cin = x.shape[1]
    return pl.pallas_call(
        _project_body,
        grid=(N // RB,),
        in_specs=[
            pl.BlockSpec((RB, cin), lambda i: (i, 0)),
            pl.BlockSpec((cin, HID), lambda i: (0, 0)),
            pl.BlockSpec((1, HID), lambda i: (0, 0)),
            pl.BlockSpec((1, HID), lambda i: (0, 0)),
        ],
        out_specs=[
            pl.BlockSpec((RB, HID), lambda i: (i, 0)),
            pl.BlockSpec((RB, 1), lambda i: (i, 0)),
            pl.BlockSpec((RB, 1), lambda i: (i, 0)),
        ],
        out_shape=[
            jax.ShapeDtypeStruct((N, HID), jnp.float32),
            jax.ShapeDtypeStruct((N, 1), jnp.float32),
            jax.ShapeDtypeStruct((N, 1), jnp.float32),
        ],
    )(x, W, asv, adv)


def _finalize_block(a0_ref, a1_ref, den_ref, b_ref):
    den = jnp.sum(den_ref[...], axis=1, keepdims=True)
    x = (a0_ref[...] + a1_ref[...]) / (den + 1e-16) + b_ref[...]
    return _elu(x)


def _finproj_body(a0_ref, a1_ref, den_ref, b_ref, w_ref, asv_ref, adv_ref,
                  h_ref, s_ref, d_ref):
    x = _finalize_block(a0_ref, a1_ref, den_ref, b_ref)
    h = jnp.dot(x, w_ref[...], preferred_element_type=jnp.float32)
    h_ref[...] = h
    s_ref[...] = jnp.sum(h * asv_ref[...], axis=1, keepdims=True)
    d_ref[...] = jnp.sum(h * adv_ref[...], axis=1, keepdims=True)


def _tc_finproj(a0, a1, denT, b, W, asv, adv):
    return pl.pallas_call(
        _finproj_body,
        grid=(N // RB,),
        in_specs=[
            pl.BlockSpec((RB, HID), lambda i: (i, 0)),
            pl.BlockSpec((RB, HID), lambda i: (i, 0)),
            pl.BlockSpec((RB, NW), lambda i: (i, 0)),
            pl.BlockSpec((1, HID), lambda i: (0, 0)),
            pl.BlockSpec((HID, HID), lambda i: (0, 0)),
            pl.BlockSpec((1, HID), lambda i: (0, 0)),
            pl.BlockSpec((1, HID), lambda i: (0, 0)),
        ],
        out_specs=[
            pl.BlockSpec((RB, HID), lambda i: (i, 0)),
            pl.BlockSpec((RB, 1), lambda i: (i, 0)),
            pl.BlockSpec((RB, 1), lambda i: (i, 0)),
        ],
        out_shape=[
            jax.ShapeDtypeStruct((N, HID), jnp.float32),
            jax.ShapeDtypeStruct((N, 1), jnp.float32),
            jax.ShapeDtypeStruct((N, 1), jnp.float32),
        ],
    )(a0, a1, denT, b, W, asv, adv)


def _final_body(a0_ref, a1_ref, den_ref, b_ref, o_ref):
    o_ref[...] = _finalize_block(a0_ref, a1_ref, den_ref, b_ref)


def _tc_final(a0, a1, denT, b):
    return pl.pallas_call(
        _final_body,
        grid=(N // RB,),
        in_specs=[
            pl.BlockSpec((RB, HID), lambda i: (i, 0)),
            pl.BlockSpec((RB, HID), lambda i: (i, 0)),
            pl.BlockSpec((RB, NW), lambda i: (i, 0)),
            pl.BlockSpec((1, HID), lambda i: (0, 0)),
        ],
        out_specs=pl.BlockSpec((RB, HID), lambda i: (i, 0)),
        out_shape=jax.ShapeDtypeStruct((N, HID), jnp.float32),
    )(a0, a1, denT, b)


def _sc_edge_body(h_hbm, asrc_hbm, adst_hbm, src_hbm, dst_hbm,
                  acc_hbm, den_hbm,
                  asrc_v, adst_v, src_v, dst_v, den_v,
                  ga_v, gb_v, sa_v, sb_v, out_v,
                  out_sh, gsem_a, gsem_b, ssem_a, ssem_b):
    c = lax.axis_index("c")
    s = lax.axis_index("s")
    wid = c * NS + s

    pltpu.sync_copy(asrc_hbm, asrc_v)
    pltpu.sync_copy(adst_hbm, adst_v)
    pltpu.sync_copy(src_hbm.at[wid], src_v)
    pltpu.sync_copy(dst_hbm.at[wid], dst_v)

    zeros16 = jnp.zeros((16,), jnp.float32)

    def _zero_den(i, carry):
        den_v[i, :] = zeros16
        return carry

    lax.fori_loop(0, DEN_R, _zero_den, 0)

    def _zero_out(i, carry):
        for q in range(HID // 16):
            out_v[i, pl.ds(q * 16, 16)] = zeros16
        return carry

    lax.fori_loop(0, RSTG, _zero_out, 0)
    for r in range(ROWS // RSTG):
        pltpu.sync_copy(out_v, out_sh.at[pl.ds(s * ROWS + r * RSTG, RSTG)])
    plsc.subcore_barrier()

    def _drain(dst_buf, dsem):
        pltpu.make_async_copy(acc_hbm.at[c].at[pl.ds(0, CH)], dst_buf,
                              dsem).wait()

    def _chunk_w(jb):
        w16s = []
        for k in range(CH // 16):
            sidx = src_v[jb, pl.ds(k * 16, 16)]
            didx = dst_v[jb, pl.ds(k * 16, 16)]
            a1 = plsc.load_gather(asrc_v, [sidx])
            a2 = plsc.load_gather(adst_v, [didx])
            e = a1 + a2
            e = jnp.where(e >= 0, e, e * NEG)
            w16 = jnp.exp(e)
            w16s.append(w16)
            plsc.addupdate_scatter(
                den_v, [lax.shift_right_logical(didx, 4), didx & 15], w16)
        return w16s

    def _chunk_scale(w16s, gbuf, sbuf):
        for k in range(CH // 16):
            w16 = w16s[k]
            for l in range(16):
                e = k * 16 + l
                wl = w16[l]
                for q in range(HID // 16):
                    sbuf[e, pl.ds(q * 16, 16)] = (
                        gbuf[e, pl.ds(q * 16, 16)] * wl)

    pltpu.async_copy(h_hbm.at[src_v.at[0]], ga_v, gsem_a)
    pltpu.async_copy(h_hbm.at[src_v.at[1]], gb_v, gsem_b)
    half = NCHUNK // 2

    def _pipe(jj, carry):
        ja = 2 * jj
        w16s = _chunk_w(ja)
        _drain(ga_v, gsem_a)

        @pl.when(jj >= 1)
        def _():
            _drain(sa_v, ssem_a)

        _chunk_scale(w16s, ga_v, sa_v)

        @pl.when(jj <= half - 1)
        def _():
            pltpu.async_copy(h_hbm.at[src_v.at[ja + 2]], ga_v, gsem_a)

        pltpu.async_copy(sa_v, out_sh.at[dst_v.at[ja]], ssem_a, add=True)

        @pl.when(jj <= half - 1)
        def _():
            jb = 2 * jj + 1
            w16s_b = _chunk_w(jb)
            _drain(gb_v, gsem_b)

            @pl.when(jj >= 1)
            def _():
                _drain(sb_v, ssem_b)

            _chunk_scale(w16s_b, gb_v, sb_v)

            @pl.when(jj <= half - 2)
            def _():
                pltpu.async_copy(h_hbm.at[src_v.at[jb + 2]], gb_v, gsem_b)

            pltpu.async_copy(sb_v, out_sh.at[dst_v.at[jb]], ssem_b, add=True)

        return carry

    lax.fori_loop(0, half + 1, _pipe, 0)
    _drain(sa_v, ssem_a)
    _drain(sb_v, ssem_b)
    plsc.subcore_barrier()

    for r in range(ROWS // RSTG):
        pltpu.sync_copy(out_sh.at[pl.ds(s * ROWS + r * RSTG, RSTG)], out_v)
        pltpu.sync_copy(out_v, acc_hbm.at[c, pl.ds(s * ROWS + r * RSTG, RSTG)])
    pltpu.sync_copy(den_v, den_hbm.at[wid])


@functools.partial(
    pl.kernel,
    out_type=(
        pltpu.HBM((NC, N, HID), jnp.float32),
        pltpu.HBM((NW, DEN_R, 16), jnp.float32),
    ),
    mesh=plsc.VectorSubcoreMesh(core_axis_name="c", subcore_axis_name="s"),
    compiler_params=pltpu.CompilerParams(use_tc_tiling_on_sc=False,
                                         needs_layout_passes=False),
    scratch_types=[
        pltpu.VMEM((N,), jnp.float32),
        pltpu.VMEM((N,), jnp.float32),
        pltpu.VMEM((NCHUNK, CH), jnp.int32),
        pltpu.VMEM((NCHUNK, CH), jnp.int32),
        pltpu.VMEM((DEN_R, 16), jnp.float32),
        pltpu.VMEM((CH, HID), jnp.float32),
        pltpu.VMEM((CH, HID), jnp.float32),
        pltpu.VMEM((CH, HID), jnp.float32),
        pltpu.VMEM((CH, HID), jnp.float32),
        pltpu.VMEM((RSTG, HID), jnp.float32),
        pltpu.VMEM_SHARED((N, HID), jnp.float32),
        pltpu.SemaphoreType.DMA,
        pltpu.SemaphoreType.DMA,
        pltpu.SemaphoreType.DMA,
        pltpu.SemaphoreType.DMA,
    ],
)
def _sc_edge(h_hbm, asrc_hbm, adst_hbm, src_hbm, dst_hbm, acc_hbm, den_hbm,
             *rest):
    _sc_edge_body(h_hbm, asrc_hbm, adst_hbm, src_hbm, dst_hbm,
                  acc_hbm, den_hbm, *rest)


def kernel(inp, edge_index, W1, a_src1, a_dst1, b1, W2, a_src2, a_dst2, b2,
           W3, a_src3, a_dst3, b3):
    src3 = edge_index[0].reshape(NW, NCHUNK, CH)
    dst3 = edge_index[1].reshape(NW, NCHUNK, CH)

    h, s, d = _tc_project(inp, W1, a_src1.reshape(1, HID),
                          a_dst1.reshape(1, HID))
    acc, den = _sc_edge(h, s.reshape(N), d.reshape(N), src3, dst3)
    denT = den.reshape(NW, N).T

    h, s, d = _tc_finproj(acc[0], acc[1], denT, b1.reshape(1, HID), W2,
                          a_src2.reshape(1, HID), a_dst2.reshape(1, HID))
    acc, den = _sc_edge(h, s.reshape(N), d.reshape(N), src3, dst3)
    denT = den.reshape(NW, N).T

    h, s, d = _tc_finproj(acc[0], acc[1], denT, b2.reshape(1, HID), W3,
                          a_src3.reshape(1, HID), a_dst3.reshape(1, HID))
    acc, den = _sc_edge(h, s.reshape(N), d.reshape(N), src3, dst3)
    denT = den.reshape(NW, N).T

    return _tc_final(acc[0], acc[1], denT, b3.reshape(1, HID))


# speedup vs baseline: 60.3596x; 1.0479x over previous
import functools

import jax
import jax.numpy as jnp
from jax import lax
from jax.experimental import pallas as pl
from jax.experimental.pallas import tpu as pltpu
from jax.experimental.pallas import tpu_sc as plsc

N = 10000
E = 320000
IN_C = 128
HID = 64
NEG = 0.2

NC = 2
NS = 16
NW = NC * NS
ET = E // NW
CH = 80
NCHUNK = ET // CH
ROWS = N // NS
RSTG = 125
CBYTES = CH * HID * 4
DEN_R = N // 16

RB = 1000


def _elu(x):
    return jnp.where(x > 0, x, jnp.exp(jnp.minimum(x, 0.0)) - 1.0)


def _project_body(x_ref, w_ref, asv_ref, adv_ref, h_ref, s_ref, d_ref):
    h = jnp.dot(x_ref[...], w_ref[...], preferred_element_type=jnp.float32)
    h_ref[...] = h
    s_ref[...] = jnp.sum(h * asv_ref[...], axis=1, keepdims=True)
    d_ref[...] = jnp.sum(h * adv_ref[...], axis=1, keepdims=True)


def _tc_project(x, W, asv, adv):
    cin = x.shape[1]
    return pl.pallas_call(
        _project_body,
        grid=(N // RB,),
        in_specs=[
            pl.BlockSpec((RB, cin), lambda i: (i, 0)),
            pl.BlockSpec((cin, HID), lambda i: (0, 0)),
            pl.BlockSpec((1, HID), lambda i: (0, 0)),
            pl.BlockSpec((1, HID), lambda i: (0, 0)),
        ],
        out_specs=[
            pl.BlockSpec((RB, HID), lambda i: (i, 0)),
            pl.BlockSpec((RB, 1), lambda i: (i, 0)),
            pl.BlockSpec((RB, 1), lambda i: (i, 0)),
        ],
        out_shape=[
            jax.ShapeDtypeStruct((N, HID), jnp.float32),
            jax.ShapeDtypeStruct((N, 1), jnp.float32),
            jax.ShapeDtypeStruct((N, 1), jnp.float32),
        ],
    )(x, W, asv, adv)


def _finalize_block(a0_ref, a1_ref, den_ref, b_ref):
    den = jnp.sum(den_ref[...], axis=1, keepdims=True)
    x = (a0_ref[...] + a1_ref[...]) / (den + 1e-16) + b_ref[...]
    return _elu(x)


def _finproj_body(a0_ref, a1_ref, den_ref, b_ref, w_ref, asv_ref, adv_ref,
                  h_ref, s_ref, d_ref):
    x = _finalize_block(a0_ref, a1_ref, den_ref, b_ref)
    h = jnp.dot(x, w_ref[...], preferred_element_type=jnp.float32)
    h_ref[...] = h
    s_ref[...] = jnp.sum(h * asv_ref[...], axis=1, keepdims=True)
    d_ref[...] = jnp.sum(h * adv_ref[...], axis=1, keepdims=True)


def _tc_finproj(a0, a1, denT, b, W, asv, adv):
    return pl.pallas_call(
        _finproj_body,
        grid=(N // RB,),
        in_specs=[
            pl.BlockSpec((RB, HID), lambda i: (i, 0)),
            pl.BlockSpec((RB, HID), lambda i: (i, 0)),
            pl.BlockSpec((RB, NW), lambda i: (i, 0)),
            pl.BlockSpec((1, HID), lambda i: (0, 0)),
            pl.BlockSpec((HID, HID), lambda i: (0, 0)),
            pl.BlockSpec((1, HID), lambda i: (0, 0)),
            pl.BlockSpec((1, HID), lambda i: (0, 0)),
        ],
        out_specs=[
            pl.BlockSpec((RB, HID), lambda i: (i, 0)),
            pl.BlockSpec((RB, 1), lambda i: (i, 0)),
            pl.BlockSpec((RB, 1), lambda i: (i, 0)),
        ],
        out_shape=[
            jax.ShapeDtypeStruct((N, HID), jnp.float32),
            jax.ShapeDtypeStruct((N, 1), jnp.float32),
            jax.ShapeDtypeStruct((N, 1), jnp.float32),
        ],
    )(a0, a1, denT, b, W, asv, adv)


def _final_body(a0_ref, a1_ref, den_ref, b_ref, o_ref):
    o_ref[...] = _finalize_block(a0_ref, a1_ref, den_ref, b_ref)


def _tc_final(a0, a1, denT, b):
    return pl.pallas_call(
        _final_body,
        grid=(N // RB,),
        in_specs=[
            pl.BlockSpec((RB, HID), lambda i: (i, 0)),
            pl.BlockSpec((RB, HID), lambda i: (i, 0)),
            pl.BlockSpec((RB, NW), lambda i: (i, 0)),
            pl.BlockSpec((1, HID), lambda i: (0, 0)),
        ],
        out_specs=pl.BlockSpec((RB, HID), lambda i: (i, 0)),
        out_shape=jax.ShapeDtypeStruct((N, HID), jnp.float32),
    )(a0, a1, denT, b)


def _sc_edge_body(h_hbm, asrc_hbm, adst_hbm, src_hbm, dst_hbm,
                  acc_hbm, den_hbm,
                  asrc_v, adst_v, src_v, dst_v, den_v,
                  ga_v, gb_v, sa_v, sb_v, out_v,
                  out_sh, gsem_a, gsem_b, ssem_a, ssem_b):
    c = lax.axis_index("c")
    s = lax.axis_index("s")
    wid = c * NS + s

    pltpu.async_copy(asrc_hbm, asrc_v, gsem_a)
    pltpu.async_copy(adst_hbm, adst_v, gsem_b)
    pltpu.async_copy(src_hbm.at[wid], src_v, ssem_a)
    pltpu.async_copy(dst_hbm.at[wid], dst_v, ssem_b)
    pltpu.make_async_copy(asrc_hbm, asrc_v, gsem_a).wait()
    pltpu.make_async_copy(adst_hbm, adst_v, gsem_b).wait()
    pltpu.make_async_copy(src_hbm.at[wid], src_v, ssem_a).wait()
    pltpu.make_async_copy(dst_hbm.at[wid], dst_v, ssem_b).wait()

    pltpu.async_copy(h_hbm.at[src_v.at[0]], ga_v, gsem_a)
    pltpu.async_copy(h_hbm.at[src_v.at[1]], gb_v, gsem_b)

    zeros16 = jnp.zeros((16,), jnp.float32)

    def _zero_den(i, carry):
        den_v[i, :] = zeros16
        return carry

    lax.fori_loop(0, DEN_R, _zero_den, 0)

    def _zero_out(i, carry):
        for q in range(HID // 16):
            out_v[i, pl.ds(q * 16, 16)] = zeros16
        return carry

    lax.fori_loop(0, RSTG, _zero_out, 0)
    for r in range(ROWS // RSTG):
        pltpu.sync_copy(out_v, out_sh.at[pl.ds(s * ROWS + r * RSTG, RSTG)])
    plsc.subcore_barrier()

    def _drain(dst_buf, dsem):
        pltpu.make_async_copy(acc_hbm.at[c].at[pl.ds(0, CH)], dst_buf,
                              dsem).wait()

    def _chunk_w(jb):
        w16s = []
        for k in range(CH // 16):
            sidx = src_v[jb, pl.ds(k * 16, 16)]
            didx = dst_v[jb, pl.ds(k * 16, 16)]
            a1 = plsc.load_gather(asrc_v, [sidx])
            a2 = plsc.load_gather(adst_v, [didx])
            e = a1 + a2
            e = jnp.where(e >= 0, e, e * NEG)
            w16 = jnp.exp(e)
            w16s.append(w16)
            plsc.addupdate_scatter(
                den_v, [lax.shift_right_logical(didx, 4), didx & 15], w16)
        return w16s

    def _chunk_scale(w16s, gbuf, sbuf):
        for k in range(CH // 16):
            w16 = w16s[k]
            for l in range(16):
                e = k * 16 + l
                wl = w16[l]
                for q in range(HID // 16):
                    sbuf[e, pl.ds(q * 16, 16)] = (
                        gbuf[e, pl.ds(q * 16, 16)] * wl)

    def _gissue(jb, gbuf, gsem):
        pltpu.async_copy(h_hbm.at[src_v.at[jb]], gbuf, gsem)

    half = NCHUNK // 2

    def _pipe(jj, carry):
        ja = 2 * jj
        w16s = _chunk_w(ja)
        _drain(ga_v, gsem_a)

        @pl.when(jj >= 1)
        def _():
            _drain(sa_v, ssem_a)

        _chunk_scale(w16s, ga_v, sa_v)

        @pl.when(jj <= half - 1)
        def _():
            _gissue(ja + 2, ga_v, gsem_a)

        pltpu.async_copy(sa_v, out_sh.at[dst_v.at[ja]], ssem_a, add=True)

        @pl.when(jj <= half - 1)
        def _():
            jb = 2 * jj + 1
            w16s_b = _chunk_w(jb)
            _drain(gb_v, gsem_b)

            @pl.when(jj >= 1)
            def _():
                _drain(sb_v, ssem_b)

            _chunk_scale(w16s_b, gb_v, sb_v)

            @pl.when(jj <= half - 2)
            def _():
                _gissue(jb + 2, gb_v, gsem_b)

            pltpu.async_copy(sb_v, out_sh.at[dst_v.at[jb]], ssem_b, add=True)

        return carry

    lax.fori_loop(0, half + 1, _pipe, 0)
    _drain(sa_v, ssem_a)
    _drain(sb_v, ssem_b)
    plsc.subcore_barrier()

    pltpu.async_copy(out_sh.at[pl.ds(s * ROWS, ROWS)],
                     acc_hbm.at[c, pl.ds(s * ROWS, ROWS)], gsem_a)
    pltpu.async_copy(den_v, den_hbm.at[wid], gsem_b)
    pltpu.make_async_copy(out_sh.at[pl.ds(s * ROWS, ROWS)],
                          acc_hbm.at[c, pl.ds(s * ROWS, ROWS)], gsem_a).wait()
    pltpu.make_async_copy(den_v, den_hbm.at[wid], gsem_b).wait()


@functools.partial(
    pl.kernel,
    out_type=(
        pltpu.HBM((NC, N, HID), jnp.float32),
        pltpu.HBM((NW, DEN_R, 16), jnp.float32),
    ),
    mesh=plsc.VectorSubcoreMesh(core_axis_name="c", subcore_axis_name="s"),
    compiler_params=pltpu.CompilerParams(use_tc_tiling_on_sc=False,
                                         needs_layout_passes=False),
    scratch_types=[
        pltpu.VMEM((N,), jnp.float32),
        pltpu.VMEM((N,), jnp.float32),
        pltpu.VMEM((NCHUNK, CH), jnp.int32),
        pltpu.VMEM((NCHUNK, CH), jnp.int32),
        pltpu.VMEM((DEN_R, 16), jnp.float32),
        pltpu.VMEM((CH, HID), jnp.float32),
        pltpu.VMEM((CH, HID), jnp.float32),
        pltpu.VMEM((CH, HID), jnp.float32),
        pltpu.VMEM((CH, HID), jnp.float32),
        pltpu.VMEM((RSTG, HID), jnp.float32),
        pltpu.VMEM_SHARED((N, HID), jnp.float32),
        pltpu.SemaphoreType.DMA,
        pltpu.SemaphoreType.DMA,
        pltpu.SemaphoreType.DMA,
        pltpu.SemaphoreType.DMA,
    ],
)
def _sc_edge(h_hbm, asrc_hbm, adst_hbm, src_hbm, dst_hbm, acc_hbm, den_hbm,
             *rest):
    _sc_edge_body(h_hbm, asrc_hbm, adst_hbm, src_hbm, dst_hbm,
                  acc_hbm, den_hbm, *rest)


def kernel(inp, edge_index, W1, a_src1, a_dst1, b1, W2, a_src2, a_dst2, b2,
           W3, a_src3, a_dst3, b3):
    src3 = edge_index[0].reshape(NW, NCHUNK, CH)
    dst3 = edge_index[1].reshape(NW, NCHUNK, CH)

    h, s, d = _tc_project(inp, W1, a_src1.reshape(1, HID),
                          a_dst1.reshape(1, HID))
    acc, den = _sc_edge(h, s.reshape(N), d.reshape(N), src3, dst3)
    denT = den.reshape(NW, N).T

    h, s, d = _tc_finproj(acc[0], acc[1], denT, b1.reshape(1, HID), W2,
                          a_src2.reshape(1, HID), a_dst2.reshape(1, HID))
    acc, den = _sc_edge(h, s.reshape(N), d.reshape(N), src3, dst3)
    denT = den.reshape(NW, N).T

    h, s, d = _tc_finproj(acc[0], acc[1], denT, b2.reshape(1, HID), W3,
                          a_src3.reshape(1, HID), a_dst3.reshape(1, HID))
    acc, den = _sc_edge(h, s.reshape(N), d.reshape(N), src3, dst3)
    denT = den.reshape(NW, N).T

    return _tc_final(acc[0], acc[1], denT, b3.reshape(1, HID))
